# baseline (speedup 1.0000x reference)
import jax
import jax.numpy as jnp
from jax import lax
from jax.experimental import pallas as pl
from jax.experimental.pallas import tpu as pltpu
from jax.experimental.pallas import tpu_sc as plsc

VOCAB = 50265
HIDDEN = 768
MAX_POS = 514
PAD = 1
EPS = 1e-05
B, L = 64, 512

LANES = 16
NV = HIDDEN // LANES
NW = 32
TPW = (B * L) // NW
ROWS_PW = TPW // L
CHUNK = 32
NCHUNK = TPW // CHUNK

_MAGIC = 0x5F3759DF


def _rsqrt_scalar(a):
    seed = jnp.int32(_MAGIC) - (lax.bitcast_convert_type(a, jnp.int32) >> 1)
    y = lax.bitcast_convert_type(seed, jnp.float32)
    half = jnp.float32(0.5) * a
    for _ in range(3):
        y = y * (jnp.float32(1.5) - half * y * y)
    return y


def _sc_body(ids_hbm, word_hbm, pos_hbm, g_hbm, b_hbm, out_hbm,
             ids_v, pos_v, wbuf, pbuf, gam_v, bet_v, stat_s,
             sem_w, sem_p, sem_o):
    wid = lax.axis_index("s") * 2 + lax.axis_index("c")
    base = wid * TPW

    pltpu.sync_copy(ids_hbm.at[pl.ds(base, TPW)], ids_v)
    pltpu.sync_copy(g_hbm, gam_v)
    pltpu.sync_copy(b_hbm, bet_v)

    one16 = jnp.full((LANES,), 1, jnp.int32)
    zero16i = jnp.zeros((LANES,), jnp.int32)
    for r in range(ROWS_PW):
        def pos_body(j, carry, r=r):
            off = pl.multiple_of(r * L + j * LANES, LANES)
            v = ids_v[pl.ds(off, LANES)]
            m = jnp.where(v != PAD, one16, zero16i)
            cs = plsc.cumsum(m) + carry
            pos_v[pl.ds(off, LANES)] = cs * m + PAD
            return carry + jnp.sum(m)
        lax.fori_loop(0, L // LANES, pos_body, jnp.int32(0))

    def gather_src(c):
        off = pl.multiple_of(c * CHUNK, 8)
        return (word_hbm.at[ids_v.at[pl.ds(off, CHUNK)]],
                pos_hbm.at[pos_v.at[pl.ds(off, CHUNK)]])

    def issue_gathers(c, q):
        srcw, srcp = gather_src(c)
        pltpu.async_copy(srcw, wbuf.at[q], sem_w.at[q])
        pltpu.async_copy(srcp, pbuf.at[q], sem_p.at[q])

    def wait_gathers(c, p):
        srcw, srcp = gather_src(c)
        pltpu.make_async_copy(srcw, wbuf.at[p], sem_w.at[p]).wait()
        pltpu.make_async_copy(srcp, pbuf.at[p], sem_p.at[p]).wait()

    def out_dst(c):
        return out_hbm.at[pl.ds(base + c * CHUNK, CHUNK)]

    zero16 = jnp.zeros((LANES,), jnp.float32)
    inv_h = jnp.float32(1.0 / HIDDEN)

    def compute(p):
        @plsc.parallel_loop(0, CHUNK, unroll=2)
        def token_body(t):
            s = [zero16, zero16, zero16, zero16]
            s2 = [zero16, zero16, zero16, zero16]
            for j in range(NV):
                ds = pl.ds(j * LANES, LANES)
                x = wbuf[p, t, ds] + pbuf[p, t, ds]
                wbuf[p, t, ds] = x
                s[j % 4] = s[j % 4] + x
                s2[j % 4] = s2[j % 4] + x * x
            sv = (s[0] + s[1]) + (s[2] + s[3])
            s2v = (s2[0] + s2[1]) + (s2[2] + s2[3])
            mean = jnp.sum(sv) * inv_h
            var = jnp.sum(s2v) * inv_h - mean * mean
            rs = _rsqrt_scalar(var + jnp.float32(EPS))
            stat_s[0, t] = rs
            stat_s[1, t] = -mean * rs

        TB = 16
        for tb in range(CHUNK // TB):
            @plsc.parallel_loop(0, NV, unroll=2)
            def slice_body(j, tb=tb):
                ds = pl.ds(j * LANES, LANES)
                g = gam_v[ds]
                b = bet_v[ds]
                for t in range(tb * TB, (tb + 1) * TB):
                    a_v = jnp.broadcast_to(stat_s[0, t], (LANES,))
                    c_v = jnp.broadcast_to(stat_s[1, t], (LANES,))
                    x = wbuf[p, t, ds]
                    wbuf[p, t, ds] = (x * a_v + c_v) * g + b

    issue_gathers(0, 0)

    def step(c, _):
        p = jnp.bitwise_and(c, 1)
        q = 1 - p

        @pl.when(c > 0)
        def _():
            pltpu.make_async_copy(wbuf.at[q], out_dst(c - 1), sem_o.at[q]).wait()

        @pl.when(c < NCHUNK - 1)
        def _():
            issue_gathers(c + 1, q)

        wait_gathers(c, p)
        compute(p)
        pltpu.async_copy(wbuf.at[p], out_dst(c), sem_o.at[p])
        return 0

    lax.fori_loop(0, NCHUNK, step, 0)

    last = NCHUNK - 1
    pltpu.make_async_copy(wbuf.at[last % 2], out_dst(last),
                          sem_o.at[last % 2]).wait()


@jax.jit
def _embed(ids_flat, word_table, pos_table, ln_gamma, ln_beta):
    mesh = plsc.VectorSubcoreMesh(core_axis_name="c", subcore_axis_name="s")
    fn = pl.kernel(
        _sc_body,
        out_type=jax.ShapeDtypeStruct((B * L, HIDDEN), jnp.float32),
        mesh=mesh,
        compiler_params=pltpu.CompilerParams(needs_layout_passes=False),
        scratch_types=[
            pltpu.VMEM((TPW,), jnp.int32),
            pltpu.VMEM((TPW,), jnp.int32),
            pltpu.VMEM((2, CHUNK, HIDDEN), jnp.float32),
            pltpu.VMEM((2, CHUNK, HIDDEN), jnp.float32),
            pltpu.VMEM((HIDDEN,), jnp.float32),
            pltpu.VMEM((HIDDEN,), jnp.float32),
            pltpu.SMEM((2, CHUNK), jnp.float32),
            pltpu.SemaphoreType.DMA((2,)),
            pltpu.SemaphoreType.DMA((2,)),
            pltpu.SemaphoreType.DMA((2,)),
        ],
    )
    return fn(ids_flat, word_table, pos_table, ln_gamma, ln_beta)


def kernel(input_ids, word_table, pos_table, ln_gamma, ln_beta):
    ids_flat = input_ids.astype(jnp.int32).reshape(B * L)
    out = _embed(ids_flat, word_table, pos_table, ln_gamma, ln_beta)
    return out.reshape(B, L, HIDDEN)


# speedup vs baseline: 1.0350x; 1.0350x over previous
import jax
import jax.numpy as jnp
from jax import lax
from jax.experimental import pallas as pl
from jax.experimental.pallas import tpu as pltpu
from jax.experimental.pallas import tpu_sc as plsc

VOCAB = 50265
HIDDEN = 768
MAX_POS = 514
PAD = 1
EPS = 1e-05
B, L = 64, 512

LANES = 16
NV = HIDDEN // LANES
NW = 32
TPW = (B * L) // NW
ROWS_PW = TPW // L
CHUNK = 32
NCHUNK = TPW // CHUNK

_MAGIC = 0x5F3759DF


def _rsqrt_scalar(a):
    seed = jnp.int32(_MAGIC) - (lax.bitcast_convert_type(a, jnp.int32) >> 1)
    y = lax.bitcast_convert_type(seed, jnp.float32)
    half = jnp.float32(0.5) * a
    for _ in range(3):
        y = y * (jnp.float32(1.5) - half * y * y)
    return y


def _sc_body(ids_hbm, word_hbm, pos_hbm, g_hbm, b_hbm, out_hbm,
             ids_v, pos_v, wbuf, pbuf, gam_v, bet_v, stat_s,
             sem_w, sem_p, sem_o):
    wid = lax.axis_index("s") * 2 + lax.axis_index("c")
    base = wid * TPW

    pltpu.sync_copy(ids_hbm.at[pl.ds(base, TPW)], ids_v)
    pltpu.sync_copy(g_hbm, gam_v)
    pltpu.sync_copy(b_hbm, bet_v)

    one16 = jnp.full((LANES,), 1, jnp.int32)
    zero16i = jnp.zeros((LANES,), jnp.int32)
    for r in range(ROWS_PW):
        def pos_body(j, carry, r=r):
            off = pl.multiple_of(r * L + j * LANES, LANES)
            v = ids_v[pl.ds(off, LANES)]
            m = jnp.where(v != PAD, one16, zero16i)
            cs = plsc.cumsum(m) + carry
            pos_v[pl.ds(off, LANES)] = cs * m + PAD
            return carry + jnp.sum(m)
        lax.fori_loop(0, L // LANES, pos_body, jnp.int32(0))

    def gather_src(c):
        off = pl.multiple_of(c * CHUNK, 8)
        return (word_hbm.at[ids_v.at[pl.ds(off, CHUNK)]],
                pos_hbm.at[pos_v.at[pl.ds(off, CHUNK)]])

    def issue_gathers(c, q):
        srcw, srcp = gather_src(c)
        pltpu.async_copy(srcw, wbuf.at[q], sem_w.at[q])
        pltpu.async_copy(srcp, pbuf.at[q], sem_p.at[q])

    def wait_gathers(c, p):
        srcw, srcp = gather_src(c)
        pltpu.make_async_copy(srcw, wbuf.at[p], sem_w.at[p]).wait()
        pltpu.make_async_copy(srcp, pbuf.at[p], sem_p.at[p]).wait()

    TB = 16
    NB = CHUNK // TB

    def out_half_pair(c, p, tb):
        src = wbuf.at[p, pl.ds(tb * TB, TB)]
        dst = out_hbm.at[pl.ds(base + c * CHUNK + tb * TB, TB)]
        return src, dst

    zero16 = jnp.zeros((LANES,), jnp.float32)
    inv_h = jnp.float32(1.0 / HIDDEN)

    def compute(c, p):
        @plsc.parallel_loop(0, CHUNK, unroll=2)
        def token_body(t):
            s = [zero16, zero16, zero16, zero16]
            s2 = [zero16, zero16, zero16, zero16]
            for j in range(NV):
                ds = pl.ds(j * LANES, LANES)
                x = wbuf[p, t, ds] + pbuf[p, t, ds]
                wbuf[p, t, ds] = x
                s[j % 4] = s[j % 4] + x
                s2[j % 4] = s2[j % 4] + x * x
            sv = (s[0] + s[1]) + (s[2] + s[3])
            s2v = (s2[0] + s2[1]) + (s2[2] + s2[3])
            mean = jnp.sum(sv) * inv_h
            var = jnp.sum(s2v) * inv_h - mean * mean
            rs = _rsqrt_scalar(var + jnp.float32(EPS))
            stat_s[0, t] = rs
            stat_s[1, t] = -mean * rs

        for tb in range(NB):
            @plsc.parallel_loop(0, NV)
            def slice_body(j, tb=tb):
                ds = pl.ds(j * LANES, LANES)
                g = gam_v[ds]
                b = bet_v[ds]
                for t in range(tb * TB, (tb + 1) * TB):
                    a_v = jnp.broadcast_to(stat_s[0, t], (LANES,))
                    c_v = jnp.broadcast_to(stat_s[1, t], (LANES,))
                    x = wbuf[p, t, ds]
                    wbuf[p, t, ds] = (x * a_v + c_v) * g + b
            src, dst = out_half_pair(c, p, tb)
            pltpu.async_copy(src, dst, sem_o.at[p])

    issue_gathers(0, 0)

    def step(c, _):
        p = jnp.bitwise_and(c, 1)
        q = 1 - p

        @pl.when(c > 0)
        def _():
            for tb in range(NB):
                src, dst = out_half_pair(c - 1, q, tb)
                pltpu.make_async_copy(src, dst, sem_o.at[q]).wait()

        @pl.when(c < NCHUNK - 1)
        def _():
            issue_gathers(c + 1, q)

        wait_gathers(c, p)
        compute(c, p)
        return 0

    lax.fori_loop(0, NCHUNK, step, 0)

    last = NCHUNK - 1
    for tb in range(NB):
        src, dst = out_half_pair(last, last % 2, tb)
        pltpu.make_async_copy(src, dst, sem_o.at[last % 2]).wait()


@jax.jit
def _embed(ids_flat, word_table, pos_table, ln_gamma, ln_beta):
    mesh = plsc.VectorSubcoreMesh(core_axis_name="c", subcore_axis_name="s")
    fn = pl.kernel(
        _sc_body,
        out_type=jax.ShapeDtypeStruct((B * L, HIDDEN), jnp.float32),
        mesh=mesh,
        compiler_params=pltpu.CompilerParams(needs_layout_passes=False),
        scratch_types=[
            pltpu.VMEM((TPW,), jnp.int32),
            pltpu.VMEM((TPW,), jnp.int32),
            pltpu.VMEM((2, CHUNK, HIDDEN), jnp.float32),
            pltpu.VMEM((2, CHUNK, HIDDEN), jnp.float32),
            pltpu.VMEM((HIDDEN,), jnp.float32),
            pltpu.VMEM((HIDDEN,), jnp.float32),
            pltpu.SMEM((2, CHUNK), jnp.float32),
            pltpu.SemaphoreType.DMA((2,)),
            pltpu.SemaphoreType.DMA((2,)),
            pltpu.SemaphoreType.DMA((2,)),
        ],
    )
    return fn(ids_flat, word_table, pos_table, ln_gamma, ln_beta)


def kernel(input_ids, word_table, pos_table, ln_gamma, ln_beta):
    ids_flat = input_ids.astype(jnp.int32).reshape(B * L)
    out = _embed(ids_flat, word_table, pos_table, ln_gamma, ln_beta)
    return out.reshape(B, L, HIDDEN)
